# Initial kernel scaffold; baseline (speedup 1.0000x reference)
import jax
import jax.numpy as jnp
from jax.experimental import pallas as pl


def kernel(x_transaction, x_account, x_device, x_ip, x_email, ei_by, ei_rev_by, ei_uses, ei_rev_uses, ei_from_ip, ei_rev_from_ip, ei_with_email, ei_rev_with_email, params):
    raise NotImplementedError("write your pallas kernel here")


# speedup vs baseline: 1.0031x; 1.0031x over previous
import jax
import jax.numpy as jnp
from jax.experimental import pallas as pl

HEADS = 4
CH = 32
HID = HEADS * CH
N_NODES = {'transaction': 100000, 'account': 50000, 'device': 20000, 'ip': 30000, 'email': 40000}
REL_META = {
    'by': ('transaction', 'account'),
    'rev_by': ('account', 'transaction'),
    'uses': ('transaction', 'device'),
    'rev_uses': ('device', 'transaction'),
    'from_ip': ('transaction', 'ip'),
    'rev_from_ip': ('ip', 'transaction'),
    'with_email': ('transaction', 'email'),
    'rev_with_email': ('email', 'transaction'),
}


def _elu_kernel(x_ref, o_ref):
    x = x_ref[...]
    o_ref[...] = jnp.where(x > 0, x, jnp.exp(jnp.minimum(x, 0.0)) - 1.0)


def _elu(x):
    n = x.shape[0]
    bn = 1000
    return pl.pallas_call(
        _elu_kernel,
        grid=(n // bn,),
        in_specs=[pl.BlockSpec((bn, HID), lambda i: (i, 0))],
        out_specs=pl.BlockSpec((bn, HID), lambda i: (i, 0)),
        out_shape=jax.ShapeDtypeStruct((n, HID), jnp.float32),
    )(x)


def _gat_conv(x_src, x_dst, src, dst, p, num_dst):
    h_src = (x_src @ p['W_src']).reshape(-1, HEADS, CH)
    h_dst = (x_dst @ p['W_dst']).reshape(-1, HEADS, CH)
    a_src = jnp.sum(h_src * p['att_src'], axis=-1)
    a_dst = jnp.sum(h_dst * p['att_dst'], axis=-1)
    alpha = a_src[src] + a_dst[dst]
    alpha = jax.nn.leaky_relu(alpha, 0.2)
    amax = jax.ops.segment_max(alpha, dst, num_segments=num_dst)
    amax = jnp.where(jnp.isfinite(amax), amax, 0.0)
    ex = jnp.exp(alpha - amax[dst])
    den = jax.ops.segment_sum(ex, dst, num_segments=num_dst)
    coef = ex / (den[dst] + 1e-16)
    msg = h_src[src] * coef[:, :, None]
    out = jax.ops.segment_sum(msg, dst, num_segments=num_dst)
    return out.reshape(num_dst, HID) + p['bias']


def kernel(x_transaction, x_account, x_device, x_ip, x_email, ei_by, ei_rev_by, ei_uses, ei_rev_uses, ei_from_ip, ei_rev_from_ip, ei_with_email, ei_rev_with_email, params):
    xs = {'transaction': x_transaction, 'account': x_account, 'device': x_device, 'ip': x_ip, 'email': x_email}
    eis = {'by': ei_by, 'rev_by': ei_rev_by, 'uses': ei_uses, 'rev_uses': ei_rev_uses,
           'from_ip': ei_from_ip, 'rev_from_ip': ei_rev_from_ip,
           'with_email': ei_with_email, 'rev_with_email': ei_rev_with_email}
    x_dict = dict(xs)
    for layer in params['layers']:
        out = {t: jnp.zeros((N_NODES[t], HID), jnp.float32) for t in N_NODES}
        for rel, (st, dt) in REL_META.items():
            ei = eis[rel]
            out[dt] = out[dt] + _gat_conv(x_dict[st], x_dict[dt], ei[0], ei[1], layer[rel], N_NODES[dt])
        x_dict = {t: _elu(v) for t, v in out.items()}
    return (x_dict['transaction'], x_dict['account'], x_dict['device'], x_dict['ip'], x_dict['email'])


# speedup vs baseline: 36.2661x; 36.1549x over previous
import functools

import jax
import jax.numpy as jnp
from jax import lax
from jax.experimental import pallas as pl
from jax.experimental.pallas import tpu as pltpu
from jax.experimental.pallas import tpu_sc as plsc

HEADS = 4
CH = 32
HID = HEADS * CH
N_NODES = {'transaction': 100000, 'account': 50000, 'device': 20000, 'ip': 30000, 'email': 40000}
REL_META = {
    'by': ('transaction', 'account'),
    'rev_by': ('account', 'transaction'),
    'uses': ('transaction', 'device'),
    'rev_uses': ('device', 'transaction'),
    'from_ip': ('transaction', 'ip'),
    'rev_from_ip': ('ip', 'transaction'),
    'with_email': ('transaction', 'email'),
    'rev_with_email': ('email', 'transaction'),
}

E = 200000
E_PAD = 200192
CHUNK = E_PAD // 16
NV = CHUNK // 16
C = 64
BM = 512


def _ranges(n_dst):
    n_pad = n_dst + 8
    rcap = 4096
    p = 2 * (-(-n_pad // (2 * rcap)))
    r = (-(-n_pad // (p * 512))) * 512
    return p, r


@functools.lru_cache(maxsize=None)
def _make_edge_kernel(r_size, ip0, npass):
    R = r_size
    NR = R + 16
    FROWS = R // 16
    mesh = plsc.VectorSubcoreMesh(core_axis_name="c", subcore_axis_name="s")

    def body(src_hbm, dst_hbm, as4_hbm, ad4_hbm, h_hbm, out_hbm,
             src_loc, dst_loc,
             idx4s, idx4d, didxloc, sfull, asflat, adflat, exbuf, rows, zdbuf, denf,
             num_sh, den_sh, sem0, sem1, sem2):
        core = lax.axis_index("c")
        sub = lax.axis_index("s")
        lane = lax.iota(jnp.int32, 16)
        mask0 = lane == 0
        mask4 = lane < 4
        fzero = jnp.zeros((16,), jnp.float32)
        izero = jnp.zeros((16,), jnp.int32)

        ebase = sub * CHUNK

        def zdb_body(i, _):
            plsc.store_scatter(zdbuf, [4 * i + lane // 4, lane % 4], fzero)
            return 0
        lax.fori_loop(0, C // 4, zdb_body, 0)

        for ip_loc in range(npass):
            lo = (2 * (ip0 + ip_loc) + core) * R
            hi = lo + R
            obase = (2 * ip_loc + core) * R

            def zr_body(i2, _):
                rows[i2 // 8, pl.ds((i2 % 8) * 16, 16)] = fzero
                return 0
            lax.fori_loop(0, C * 8, zr_body, 0)
            zbase = sub * FROWS
            off = 0
            while off < FROWS:
                sz = min(C, FROWS - off)
                pltpu.sync_copy(rows.at[pl.ds(0, sz)], num_sh.at[pl.ds(zbase + off, sz)])
                pltpu.sync_copy(zdbuf.at[pl.ds(0, sz)], den_sh.at[pl.ds(zbase + off, sz)])
                off += sz
            plsc.subcore_barrier()

            eoff = 0
            while eoff < CHUNK:
                esz = min(2048, CHUNK - eoff)
                pltpu.sync_copy(src_hbm.at[pl.ds(ebase + eoff, esz)],
                                src_loc.at[pl.ds(eoff, esz)])
                pltpu.sync_copy(dst_hbm.at[pl.ds(ebase + eoff, esz)],
                                dst_loc.at[pl.ds(eoff, esz)])
                eoff += esz

            def cmp_body(v, o):
                d = dst_loc[pl.ds(v * 16, 16)]
                s = src_loc[pl.ds(v * 16, 16)]
                m = (d >= lo) & (d < hi)
                plsc.store_compressed(dst_loc.at[pl.ds(o, 16)], d, mask=m)
                plsc.store_compressed(src_loc.at[pl.ds(o, 16)], s, mask=m)
                pc = plsc.all_reduce_population_count(m)
                return o + pc[0]
            ntot = lax.fori_loop(0, NV, cmp_body, 0)

            for j in range(8):
                src_loc[pl.ds(ntot + j * 16, 16)] = lane % 8
                dst_loc[pl.ds(ntot + j * 16, 16)] = (lo + R) + lane
            ng = (ntot + (C - 1)) // C

            def ch_body(g, _):
                gc = g * C

                def ib_body(k, _):
                    ev = gc + 4 * k + lane // 4
                    sv = plsc.load_gather(src_loc, [ev])
                    dv = plsc.load_gather(dst_loc, [ev])
                    idx4s[k // 4, pl.ds((k % 4) * 16, 16)] = 4 * sv + lane % 4
                    idx4d[k // 4, pl.ds((k % 4) * 16, 16)] = 4 * dv + lane % 4
                    return 0
                lax.fori_loop(0, 4 * C // 16, ib_body, 0)

                def dl_body(k, _):
                    dv = dst_loc[pl.ds(gc + k * 16, 16)]
                    sv = src_loc[pl.ds(gc + k * 16, 16)]
                    didxloc[pl.ds(k * 16, 16)] = dv - lo
                    sfull[0, pl.ds(k * 16, 16)] = sv
                    return 0
                lax.fori_loop(0, C // 16, dl_body, 0)

                cps = []
                for j in range(4):
                    cps.append(pltpu.async_copy(
                        as4_hbm.at[idx4s.at[j]], asflat.at[j], sem0))
                    cps.append(pltpu.async_copy(
                        ad4_hbm.at[idx4d.at[j]], adflat.at[j], sem1))
                cps.append(pltpu.async_copy(
                    h_hbm.at[sfull.at[0]], rows, sem2))
                for cp in cps:
                    cp.wait()

                def grp_body(k, _):
                    a = (asflat[k // 4, pl.ds((k % 4) * 16, 16)]
                         + adflat[k // 4, pl.ds((k % 4) * 16, 16)])
                    ex = jnp.exp(jnp.maximum(a, 0.2 * a))
                    plsc.store_scatter(
                        exbuf, [4 * k + lane // 4, lane % 4], ex)
                    for j in range(4):
                        r = 4 * k + j
                        for h in range(HEADS):
                            b = fzero + ex[4 * j + h]
                            for q in range(2):
                                col = h * CH + q * 16
                                rows[r, pl.ds(col, 16)] = rows[r, pl.ds(col, 16)] * b
                    return 0
                lax.fori_loop(0, 4 * C // 16, grp_body, 0)

                pltpu.sync_copy(exbuf, den_sh.at[didxloc], add=True)
                pltpu.sync_copy(rows, num_sh.at[didxloc], add=True)
                return 0
            lax.fori_loop(0, ng, ch_body, 0)
            plsc.subcore_barrier()

            fbase = sub * FROWS
            foff = 0
            while foff < FROWS:
                sz = min(C, FROWS - foff)
                rb = fbase + foff
                pltpu.sync_copy(num_sh.at[pl.ds(rb, sz)], rows.at[pl.ds(0, sz)])
                pltpu.sync_copy(den_sh.at[pl.ds(rb, sz)], denf.at[pl.ds(0, sz)])

                def f_body(k, _):
                    dvec = plsc.load_gather(denf, [4 * k + lane // 4, lane % 4]) + 1e-16
                    for j in range(4):
                        r = 4 * k + j
                        for h in range(HEADS):
                            b = fzero + dvec[4 * j + h]
                            for q in range(2):
                                col = h * CH + q * 16
                                rows[r, pl.ds(col, 16)] = rows[r, pl.ds(col, 16)] / b
                    return 0
                lax.fori_loop(0, sz // 4, f_body, 0)
                pltpu.sync_copy(rows.at[pl.ds(0, sz)], out_hbm.at[pl.ds(obase + rb, sz)])
                foff += sz
            plsc.subcore_barrier()

    return pl.kernel(
        body,
        out_type=jax.ShapeDtypeStruct((npass * 2 * R, HID), jnp.float32),
        mesh=mesh,
        compiler_params=pltpu.CompilerParams(needs_layout_passes=False),
        scratch_types=[
            pltpu.VMEM((CHUNK + 2 * C,), jnp.int32),
            pltpu.VMEM((CHUNK + 2 * C,), jnp.int32),
            pltpu.VMEM((4, C), jnp.int32),
            pltpu.VMEM((4, C), jnp.int32),
            pltpu.VMEM((C,), jnp.int32),
            pltpu.VMEM((1, C), jnp.int32),
            pltpu.VMEM((4, C), jnp.float32),
            pltpu.VMEM((4, C), jnp.float32),
            pltpu.VMEM((C, 4), jnp.float32),
            pltpu.VMEM((C, HID), jnp.float32),
            pltpu.VMEM((C, 4), jnp.float32),
            pltpu.VMEM((C, 4), jnp.float32),
            pltpu.VMEM_SHARED((NR, HID), jnp.float32),
            pltpu.VMEM_SHARED((NR, 4), jnp.float32),
            pltpu.SemaphoreType.DMA,
            pltpu.SemaphoreType.DMA,
            pltpu.SemaphoreType.DMA,
        ],
        name=f"gat_edge_r{R}_i{ip0}_n{npass}",
    )


def _mm_call(x_pad, w_cat, w_a, s):
    n_pad = x_pad.shape[0]
    din = x_pad.shape[1]
    acols = w_a.shape[1]

    def body(x_ref, w_ref, wa_ref, *outs):
        x = x_ref[...]
        hs = jnp.dot(x, w_ref[...], preferred_element_type=jnp.float32)
        for r in range(s):
            outs[r][...] = hs[:, r * HID:(r + 1) * HID]
        outs[s][...] = jnp.dot(x, wa_ref[...], preferred_element_type=jnp.float32)

    return pl.pallas_call(
        body,
        grid=(n_pad // BM,),
        in_specs=[
            pl.BlockSpec((BM, din), lambda i: (i, 0)),
            pl.BlockSpec((din, HID * s), lambda i: (0, 0)),
            pl.BlockSpec((din, acols), lambda i: (0, 0)),
        ],
        out_specs=[pl.BlockSpec((BM, HID), lambda i: (i, 0))] * s
        + [pl.BlockSpec((BM, acols), lambda i: (i, 0))],
        out_shape=[jax.ShapeDtypeStruct((n_pad, HID), jnp.float32)] * s
        + [jax.ShapeDtypeStruct((n_pad, acols), jnp.float32)],
    )(x_pad, w_cat, w_a)


def _combine_call(convs, bias, n):
    k = len(convs)
    nblk = -(-n // BM)

    def body(*refs):
        ins, bias_ref, o_ref = refs[:k], refs[k], refs[k + 1]
        acc = ins[0][...]
        for r in ins[1:]:
            acc = acc + r[...]
        acc = acc + bias_ref[...]
        o_ref[...] = jnp.where(acc > 0, acc, jnp.exp(jnp.minimum(acc, 0.0)) - 1.0)

    return pl.pallas_call(
        body,
        grid=(nblk,),
        in_specs=[pl.BlockSpec((BM, HID), lambda i: (i, 0))] * k
        + [pl.BlockSpec((1, HID), lambda i: (0, 0))],
        out_specs=pl.BlockSpec((BM, HID), lambda i: (i, 0)),
        out_shape=jax.ShapeDtypeStruct((n, HID), jnp.float32),
    )(*convs, bias)


def _fold_att(w, att):
    din = w.shape[0]
    wt = jnp.einsum('dhc,hc->dh', w.reshape(din, HEADS, CH), att)
    return jnp.pad(wt, ((0, 0), (0, 16 - HEADS)))


def _pad_rows(x, m):
    n = x.shape[0]
    n_pad = (-(-n // m)) * m
    return jnp.pad(x, ((0, n_pad - n), (0, 0)))


def kernel(x_transaction, x_account, x_device, x_ip, x_email, ei_by, ei_rev_by, ei_uses, ei_rev_uses, ei_from_ip, ei_rev_from_ip, ei_with_email, ei_rev_with_email, params):
    xs = {'transaction': x_transaction, 'account': x_account, 'device': x_device, 'ip': x_ip, 'email': x_email}
    eis = {'by': ei_by, 'rev_by': ei_rev_by, 'uses': ei_uses, 'rev_uses': ei_rev_uses,
           'from_ip': ei_from_ip, 'rev_from_ip': ei_rev_from_ip,
           'with_email': ei_with_email, 'rev_with_email': ei_rev_with_email}

    pad_idx = jnp.arange(E_PAD - E, dtype=jnp.int32)
    src_pad, dst_pad = {}, {}
    for rel, (st, dt) in REL_META.items():
        ei = eis[rel].astype(jnp.int32)
        src_pad[rel] = jnp.concatenate([ei[0], pad_idx % 8])
        dst_pad[rel] = jnp.concatenate([ei[1], N_NODES[dt] + (pad_idx % 8)])

    ablocks = {t: [] for t in N_NODES}
    for rel, (st, dt) in REL_META.items():
        ablocks[st].append((rel, 'src'))
        ablocks[dt].append((rel, 'dst'))
    srels = {t: [rel for rel, (st, _) in REL_META.items() if st == t] for t in N_NODES}
    drels = {t: [rel for rel, (_, dt) in REL_META.items() if dt == t] for t in N_NODES}

    x_dict = dict(xs)
    for layer in params['layers']:
        h_tab = {}
        a_tab = {}
        for t in N_NODES:
            s = len(srels[t])
            w_cat = jnp.concatenate([layer[r]['W_src'] for r in srels[t]], axis=1)
            w_a = jnp.concatenate(
                [_fold_att(layer[r]['W_src' if role == 'src' else 'W_dst'],
                           layer[r]['att_src' if role == 'src' else 'att_dst'])
                 for r, role in ablocks[t]], axis=1)
            outs = _mm_call(_pad_rows(x_dict[t], BM), w_cat, w_a, s)
            for j, r in enumerate(srels[t]):
                h_tab[r] = outs[j]
            a_tab[t] = outs[s]

        conv = {}
        for rel, (st, dt) in REL_META.items():
            i_src = ablocks[st].index((rel, 'src'))
            i_dst = ablocks[dt].index((rel, 'dst'))
            as4 = a_tab[st][:, 16 * i_src:16 * i_src + 4].reshape(-1)
            ad4 = a_tab[dt][:, 16 * i_dst:16 * i_dst + 4].reshape(-1)
            P, R = _ranges(N_NODES[dt])
            npc = P // 2
            pieces = []
            ip0 = 0
            while ip0 < npc:
                np_call = min(4, npc - ip0)
                pieces.append(_make_edge_kernel(R, ip0, np_call)(
                    src_pad[rel], dst_pad[rel], as4, ad4, h_tab[rel]))
                ip0 += np_call
            conv[rel] = (jnp.concatenate(pieces, axis=0)
                         if len(pieces) > 1 else pieces[0])

        x_dict = {}
        for t in N_NODES:
            bias = sum(layer[r]['bias'] for r in drels[t]).reshape(1, HID)
            x_dict[t] = _combine_call([conv[r] for r in drels[t]], bias, N_NODES[t])

    return (x_dict['transaction'], x_dict['account'], x_dict['device'], x_dict['ip'], x_dict['email'])
